# Initial kernel scaffold; baseline (speedup 1.0000x reference)
import jax
import jax.numpy as jnp
from jax.experimental import pallas as pl


def kernel(x, edge_attr, W1, We1, as1, ad1, ae1, b1, W2, We2, as2, ad2, ae2, b2, A1, bA1, A2, bA2, N1, bN1, N2, bN2, N3, bN3, edge_index):
    raise NotImplementedError("write your pallas kernel here")


# speedup vs baseline: 29.5425x; 29.5425x over previous
import functools

import jax
import jax.numpy as jnp
from jax import lax
from jax.experimental import pallas as pl
from jax.experimental.pallas import tpu as pltpu
from jax.experimental.pallas import tpu_sc as plsc

N = 10000
E = 320000
DF = 128
DE = 16
H = 64
NACT = 3

NPAD = 10240
NW = 32
EW = E // NW
K = 80
NCH = EW // K
L = 16
RPT = NPAD // 16


_sc_mesh = plsc.VectorSubcoreMesh(core_axis_name="c", subcore_axis_name="s")


@functools.partial(
    pl.kernel,
    out_type=(
        jax.ShapeDtypeStruct((2, NPAD, H), jnp.float32),
        jax.ShapeDtypeStruct((2, NPAD), jnp.float32),
    ),
    mesh=_sc_mesh,
    compiler_params=pltpu.CompilerParams(needs_layout_passes=False,
                                         use_tc_tiling_on_sc=False),
    scratch_types=[
        pltpu.VMEM((NCH, K), jnp.int32),
        pltpu.VMEM((NCH, K), jnp.int32),
        pltpu.VMEM((NCH, K), jnp.float32),
        pltpu.VMEM((NCH, K), jnp.float32),
        pltpu.VMEM((N,), jnp.float32),
        pltpu.VMEM((N,), jnp.float32),
        pltpu.VMEM((K, H), jnp.float32),
        pltpu.VMEM_SHARED((NPAD, H), jnp.float32),
        pltpu.VMEM_SHARED((NPAD,), jnp.float32),
        pltpu.SemaphoreType.DMA,
    ],
)
def _gat_edge_sc(src_hbm, dst_hbm, ae_hbm, asrc_hbm, adst_hbm, h_hbm,
                 z2_hbm, z1_hbm, out_hbm, s_hbm,
                 src_v, dst_v, ae_v, e_v, asrc_v, adst_v, rows_v,
                 out_sh, s_sh, sem):
    cid = lax.axis_index("c")
    sid = lax.axis_index("s")
    wid = cid * 16 + sid

    pltpu.sync_copy(src_hbm.at[wid], src_v)
    pltpu.sync_copy(dst_hbm.at[wid], dst_v)
    pltpu.sync_copy(ae_hbm.at[wid], ae_v)
    pltpu.sync_copy(asrc_hbm, asrc_v)
    pltpu.sync_copy(adst_hbm, adst_v)

    r0 = sid * RPT
    pltpu.sync_copy(z2_hbm.at[pl.ds(r0, RPT)], out_sh.at[pl.ds(r0, RPT)])
    pltpu.sync_copy(z1_hbm.at[pl.ds(r0, RPT)], s_sh.at[pl.ds(r0, RPT)])
    plsc.subcore_barrier()

    def chunk(j, carry):
        for v in range(K // L):
            sidx = src_v[j, pl.ds(v * L, L)]
            didx = dst_v[j, pl.ds(v * L, L)]
            ae = ae_v[j, pl.ds(v * L, L)]
            lg = (plsc.load_gather(asrc_v, [sidx])
                  + plsc.load_gather(adst_v, [didx]) + ae)
            lg = jnp.where(lg >= 0.0, lg, 0.2 * lg)
            e_v[j, pl.ds(v * L, L)] = jnp.exp(lg)
        pltpu.async_copy(h_hbm.at[src_v.at[j]], rows_v, sem).wait()
        for g in range(K // L):
            ev = e_v[j, pl.ds(g * L, L)]
            for t in range(L):
                i = g * L + t
                ei = ev[t]
                for q in range(H // L):
                    rows_v[i, pl.ds(q * L, L)] = rows_v[i, pl.ds(q * L, L)] * ei
        pltpu.sync_copy(rows_v, out_sh.at[dst_v.at[j]], add=True)
        pltpu.sync_copy(e_v.at[j], s_sh.at[dst_v.at[j]], add=True)
        return carry

    lax.fori_loop(0, NCH, chunk, 0)
    plsc.subcore_barrier()

    pltpu.sync_copy(out_sh.at[pl.ds(r0, RPT)], out_hbm.at[cid, pl.ds(r0, RPT)])
    pltpu.sync_copy(s_sh.at[pl.ds(r0, RPT)], s_hbm.at[cid, pl.ds(r0, RPT)])


def _edge_prep_body(ea_ref, we1_ref, ae1_ref, we2_ref, ae2_ref, o1_ref, o2_ref):
    ea = ea_ref[...]
    row = lax.broadcasted_iota(jnp.int32, (DF, 8), 0)
    col = lax.broadcasted_iota(jnp.int32, (DF, 8), 1)
    for we_ref, ae_ref, o_ref in ((we1_ref, ae1_ref, o1_ref),
                                  (we2_ref, ae2_ref, o2_ref)):
        wv = jnp.dot(we_ref[...], ae_ref[...],
                     preferred_element_type=jnp.float32)
        wv8 = jnp.broadcast_to(wv.reshape(1, DE), (8, DE)).reshape(DF, 1)
        wbd = jnp.where(row // DE == col, wv8, 0.0)
        o_ref[...] = jnp.dot(ea, wbd, preferred_element_type=jnp.float32)


def _node_prep_body(x_ref, w_ref, as_ref, ad_ref, h_ref, asrc_ref, adst_ref):
    h = jnp.dot(x_ref[...], w_ref[...], preferred_element_type=jnp.float32)
    h_ref[...] = h
    asrc_ref[...] = jnp.dot(h, as_ref[...], preferred_element_type=jnp.float32)
    adst_ref[...] = jnp.dot(h, ad_ref[...], preferred_element_type=jnp.float32)


def _combine_body(num_ref, s_ref, b_ref, w2_ref, as_ref, ad_ref,
                  h2_ref, asrc_ref, adst_ref):
    num = num_ref[0] + num_ref[1]
    s = s_ref[0] + s_ref[1]
    h1 = jnp.maximum(num / (s + 1e-16) + b_ref[...], 0.0)
    h2 = jnp.dot(h1, w2_ref[...], preferred_element_type=jnp.float32)
    h2_ref[...] = h2
    asrc_ref[...] = jnp.dot(h2, as_ref[...], preferred_element_type=jnp.float32)
    adst_ref[...] = jnp.dot(h2, ad_ref[...], preferred_element_type=jnp.float32)


def _heads_body(num_ref, s_ref, b_ref, a1_ref, ba1_ref, a2_ref, ba2_ref,
                n1_ref, bn1_ref, n2_ref, bn2_ref, n3_ref, bn3_ref,
                ap_ref, ns_ref):
    num = num_ref[0, :N, :] + num_ref[1, :N, :]
    s = s_ref[0, :N, :] + s_ref[1, :N, :]
    h = num / (s + 1e-16) + b_ref[...]
    emb = jnp.mean(h, axis=0, keepdims=True)
    a = jnp.dot(emb, a1_ref[...], preferred_element_type=jnp.float32) + ba1_ref[...]
    a = jnp.where(a >= 0.0, a, 0.01 * a)
    a = jnp.dot(a, a2_ref[...], preferred_element_type=jnp.float32) + ba2_ref[...]
    a = jnp.where(a >= 0.0, a, 0.01 * a)
    a = a - jnp.max(a, axis=-1, keepdims=True)
    ea = jnp.exp(a)
    ap_ref[...] = ea / jnp.sum(ea, axis=-1, keepdims=True)
    z = jnp.dot(h, n1_ref[...], preferred_element_type=jnp.float32) + bn1_ref[...]
    z = jnp.where(z >= 0.0, z, 0.01 * z)
    z = jnp.dot(z, n2_ref[...], preferred_element_type=jnp.float32) + bn2_ref[...]
    z = jnp.where(z >= 0.0, z, 0.01 * z)
    z = jnp.dot(z, n3_ref[...], preferred_element_type=jnp.float32) + bn3_ref[...]
    ns_ref[...] = 1.0 / (1.0 + jnp.exp(-z))


_EB = 8
_edge_prep = pl.pallas_call(
    _edge_prep_body,
    grid=(_EB,),
    in_specs=[
        pl.BlockSpec((E // (8 * _EB), DF), lambda i: (i, 0)),
        pl.BlockSpec((DE, H), lambda i: (0, 0)),
        pl.BlockSpec((H, 1), lambda i: (0, 0)),
        pl.BlockSpec((DE, H), lambda i: (0, 0)),
        pl.BlockSpec((H, 1), lambda i: (0, 0)),
    ],
    out_specs=(pl.BlockSpec((E // (8 * _EB), 8), lambda i: (i, 0)),
               pl.BlockSpec((E // (8 * _EB), 8), lambda i: (i, 0))),
    out_shape=(jax.ShapeDtypeStruct((E // 8, 8), jnp.float32),
               jax.ShapeDtypeStruct((E // 8, 8), jnp.float32)),
)

_node_prep = pl.pallas_call(
    _node_prep_body,
    out_shape=(jax.ShapeDtypeStruct((N, H), jnp.float32),
               jax.ShapeDtypeStruct((N, 1), jnp.float32),
               jax.ShapeDtypeStruct((N, 1), jnp.float32)),
)

_combine = pl.pallas_call(
    _combine_body,
    out_shape=(jax.ShapeDtypeStruct((NPAD, H), jnp.float32),
               jax.ShapeDtypeStruct((NPAD, 1), jnp.float32),
               jax.ShapeDtypeStruct((NPAD, 1), jnp.float32)),
)

_heads = pl.pallas_call(
    _heads_body,
    out_shape=(jax.ShapeDtypeStruct((1, NACT), jnp.float32),
               jax.ShapeDtypeStruct((N, 1), jnp.float32)),
)


def kernel(x, edge_attr, W1, We1, as1, ad1, ae1, b1, W2, We2, as2, ad2, ae2,
           b2, A1, bA1, A2, bA2, N1, bN1, N2, bN2, N3, bN3, edge_index):
    src3 = edge_index[0].reshape(NW, NCH, K)
    dst3 = edge_index[1].reshape(NW, NCH, K)
    z2 = jnp.zeros((NPAD, H), jnp.float32)
    z1 = jnp.zeros((NPAD,), jnp.float32)

    ae_l1, ae_l2 = _edge_prep(edge_attr.reshape(E // 8, DF), We1,
                              ae1.reshape(H, 1), We2, ae2.reshape(H, 1))
    ae_l1 = ae_l1.reshape(NW, NCH, K)
    ae_l2 = ae_l2.reshape(NW, NCH, K)

    h1, asrc1, adst1 = _node_prep(x, W1, as1.reshape(H, 1), ad1.reshape(H, 1))
    out1, s1 = _gat_edge_sc(src3, dst3, ae_l1, asrc1.reshape(N),
                            adst1.reshape(N), h1, z2, z1)

    h2, asrc2, adst2 = _combine(out1, s1.reshape(2, NPAD, 1),
                                b1.reshape(1, H), W2,
                                as2.reshape(H, 1), ad2.reshape(H, 1))
    out2, s2 = _gat_edge_sc(src3, dst3, ae_l2, asrc2[:N].reshape(N),
                            adst2[:N].reshape(N), h2[:N], z2, z1)

    action_prob, node_scores = _heads(
        out2, s2.reshape(2, NPAD, 1), b2.reshape(1, H),
        A1, bA1.reshape(1, H), A2, bA2.reshape(1, NACT),
        N1, bN1.reshape(1, H), N2, bN2.reshape(1, H),
        N3, bN3.reshape(1, 1))
    return (action_prob.reshape(NACT), node_scores)


# speedup vs baseline: 39.5895x; 1.3401x over previous
import functools

import jax
import jax.numpy as jnp
from jax import lax
from jax.experimental import pallas as pl
from jax.experimental.pallas import tpu as pltpu
from jax.experimental.pallas import tpu_sc as plsc

N = 10000
E = 320000
DF = 128
DE = 16
H = 64
NACT = 3

NPAD = 10240
NW = 32
EW = E // NW
K = 80
NCH = EW // K
L = 16
RPT = NPAD // 16


_sc_mesh = plsc.VectorSubcoreMesh(core_axis_name="c", subcore_axis_name="s")


@functools.partial(
    pl.kernel,
    out_type=(
        jax.ShapeDtypeStruct((2, NPAD, H), jnp.float32),
        jax.ShapeDtypeStruct((2, NPAD), jnp.float32),
    ),
    mesh=_sc_mesh,
    compiler_params=pltpu.CompilerParams(needs_layout_passes=False,
                                         use_tc_tiling_on_sc=False),
    scratch_types=[
        pltpu.VMEM((EW,), jnp.int32),
        pltpu.VMEM((EW,), jnp.int32),
        pltpu.VMEM((EW,), jnp.float32),
        pltpu.VMEM((EW,), jnp.float32),
        pltpu.VMEM((NPAD,), jnp.float32),
        pltpu.VMEM((NPAD,), jnp.float32),
        pltpu.VMEM((K,), jnp.int32),
        pltpu.VMEM((K,), jnp.int32),
        pltpu.VMEM((K,), jnp.int32),
        pltpu.VMEM((K, H), jnp.float32),
        pltpu.VMEM((K, H), jnp.float32),
        pltpu.VMEM_SHARED((NPAD, H), jnp.float32),
        pltpu.VMEM_SHARED((NPAD,), jnp.float32),
        pltpu.SemaphoreType.DMA,
        pltpu.SemaphoreType.DMA,
    ],
)
def _gat_edge_sc(ei_hbm, ae_hbm, asrc_hbm, adst_hbm, h_hbm,
                 z2_hbm, z1_hbm, out_hbm, s_hbm,
                 src_v, dst_v, ae_v, e_v, asrc_v, adst_v, dstrow_v,
                 srow0_v, srow1_v, rows0_v, rows1_v, out_sh, s_sh,
                 sem0, sem1):
    cid = lax.axis_index("c")
    sid = lax.axis_index("s")
    wid = cid * 16 + sid
    b0 = wid * EW

    pltpu.sync_copy(ei_hbm.at[0, pl.ds(b0, EW)], src_v)
    pltpu.sync_copy(ei_hbm.at[1, pl.ds(b0, EW)], dst_v)
    pltpu.sync_copy(ae_hbm.at[pl.ds(b0, EW)], ae_v)
    pltpu.sync_copy(asrc_hbm, asrc_v)
    pltpu.sync_copy(adst_hbm, adst_v)

    r0 = sid * RPT
    pltpu.sync_copy(z2_hbm.at[pl.ds(r0, RPT)], out_sh.at[pl.ds(r0, RPT)])
    pltpu.sync_copy(z1_hbm.at[pl.ds(r0, RPT)], s_sh.at[pl.ds(r0, RPT)])
    plsc.subcore_barrier()

    def p1(m, carry):
        sidx = src_v[pl.ds(m * L, L)]
        didx = dst_v[pl.ds(m * L, L)]
        ae = ae_v[pl.ds(m * L, L)]
        lg = (plsc.load_gather(asrc_v, [sidx])
              + plsc.load_gather(adst_v, [didx]) + ae)
        lg = jnp.where(lg >= 0.0, lg, 0.2 * lg)
        e_v[pl.ds(m * L, L)] = jnp.exp(lg)
        return carry

    lax.fori_loop(0, EW // L, p1, 0)

    def _gather(j, srow_v, rows_v, sem):
        for v in range(K // L):
            srow_v[pl.ds(v * L, L)] = src_v[pl.ds(j * K + v * L, L)]
        pltpu.async_copy(h_hbm.at[srow_v], rows_v, sem)

    def _process(j, srow_v, rows_v, sem, prefetch):
        if prefetch is not None:
            pj, psrow_v, prows_v, psem = prefetch

            @pl.when(pj < NCH)
            def _():
                _gather(pj, psrow_v, prows_v, psem)

        pltpu.make_async_copy(h_hbm.at[srow_v], rows_v, sem).wait()
        for v in range(K // L):
            dstrow_v[pl.ds(v * L, L)] = dst_v[pl.ds(j * K + v * L, L)]
        for g in range(K // L):
            ev = e_v[pl.ds(j * K + g * L, L)]
            for t in range(L):
                i = g * L + t
                ei = ev[t]
                for q in range(H // L):
                    rows_v[i, pl.ds(q * L, L)] = (
                        rows_v[i, pl.ds(q * L, L)] * ei)
        pltpu.sync_copy(rows_v, out_sh.at[dstrow_v], add=True)
        pltpu.sync_copy(e_v.at[pl.ds(j * K, K)], s_sh.at[dstrow_v], add=True)

    _gather(0, srow0_v, rows0_v, sem0)

    def pair(p, carry):
        jj = p * 2
        _process(jj, srow0_v, rows0_v, sem0, (jj + 1, srow1_v, rows1_v, sem1))
        _process(jj + 1, srow1_v, rows1_v, sem1,
                 (jj + 2, srow0_v, rows0_v, sem0))
        return carry

    lax.fori_loop(0, NCH // 2, pair, 0)
    _process(NCH - 1, srow0_v, rows0_v, sem0, None)
    plsc.subcore_barrier()

    pltpu.sync_copy(out_sh.at[pl.ds(r0, RPT)], out_hbm.at[cid, pl.ds(r0, RPT)])
    pltpu.sync_copy(s_sh.at[pl.ds(r0, RPT)], s_hbm.at[cid, pl.ds(r0, RPT)])


def _edge_prep_body(ea_ref, we1_ref, ae1_ref, we2_ref, ae2_ref, o1_ref, o2_ref):
    ea = ea_ref[...]
    row = lax.broadcasted_iota(jnp.int32, (DF, 8), 0)
    col = lax.broadcasted_iota(jnp.int32, (DF, 8), 1)
    for we_ref, ae_ref, o_ref in ((we1_ref, ae1_ref, o1_ref),
                                  (we2_ref, ae2_ref, o2_ref)):
        wv = jnp.dot(we_ref[...], ae_ref[...],
                     preferred_element_type=jnp.float32)
        wv8 = jnp.broadcast_to(wv.reshape(1, DE), (8, DE)).reshape(DF, 1)
        wbd = jnp.where(row // DE == col, wv8, 0.0)
        o_ref[...] = jnp.dot(ea, wbd, preferred_element_type=jnp.float32)


def _node_prep_body(x_ref, w_ref, as_ref, ad_ref, h_ref, asrc_ref, adst_ref):
    h = jnp.dot(x_ref[...], w_ref[...], preferred_element_type=jnp.float32)
    h_ref[pl.ds(0, N), :] = h
    h_ref[pl.ds(N, NPAD - N), :] = jnp.zeros((NPAD - N, H), jnp.float32)
    asrc = jnp.dot(h, as_ref[...], preferred_element_type=jnp.float32)
    adst = jnp.dot(h, ad_ref[...], preferred_element_type=jnp.float32)
    asrc_ref[pl.ds(0, N), :] = asrc
    asrc_ref[pl.ds(N, NPAD - N), :] = jnp.zeros((NPAD - N, 1), jnp.float32)
    adst_ref[pl.ds(0, N), :] = adst
    adst_ref[pl.ds(N, NPAD - N), :] = jnp.zeros((NPAD - N, 1), jnp.float32)


def _combine_body(num_ref, s_ref, b_ref, w2_ref, as_ref, ad_ref,
                  h2_ref, asrc_ref, adst_ref):
    num = num_ref[0] + num_ref[1]
    s = s_ref[0] + s_ref[1]
    h1 = jnp.maximum(num / (s + 1e-16) + b_ref[...], 0.0)
    h2 = jnp.dot(h1, w2_ref[...], preferred_element_type=jnp.float32)
    h2_ref[...] = h2
    asrc_ref[...] = jnp.dot(h2, as_ref[...], preferred_element_type=jnp.float32)
    adst_ref[...] = jnp.dot(h2, ad_ref[...], preferred_element_type=jnp.float32)


def _heads_body(num_ref, s_ref, b_ref, a1_ref, ba1_ref, a2_ref, ba2_ref,
                n1_ref, bn1_ref, n2_ref, bn2_ref, n3_ref, bn3_ref,
                ap_ref, ns_ref):
    num = num_ref[0, :N, :] + num_ref[1, :N, :]
    s = s_ref[0, :N, :] + s_ref[1, :N, :]
    h = num / (s + 1e-16) + b_ref[...]
    emb = jnp.mean(h, axis=0, keepdims=True)
    a = jnp.dot(emb, a1_ref[...], preferred_element_type=jnp.float32) + ba1_ref[...]
    a = jnp.where(a >= 0.0, a, 0.01 * a)
    a = jnp.dot(a, a2_ref[...], preferred_element_type=jnp.float32) + ba2_ref[...]
    a = jnp.where(a >= 0.0, a, 0.01 * a)
    a = a - jnp.max(a, axis=-1, keepdims=True)
    ea = jnp.exp(a)
    ap_ref[...] = ea / jnp.sum(ea, axis=-1, keepdims=True)
    z = jnp.dot(h, n1_ref[...], preferred_element_type=jnp.float32) + bn1_ref[...]
    z = jnp.where(z >= 0.0, z, 0.01 * z)
    z = jnp.dot(z, n2_ref[...], preferred_element_type=jnp.float32) + bn2_ref[...]
    z = jnp.where(z >= 0.0, z, 0.01 * z)
    z = jnp.dot(z, n3_ref[...], preferred_element_type=jnp.float32) + bn3_ref[...]
    ns_ref[...] = 1.0 / (1.0 + jnp.exp(-z))


_EB = 8
_edge_prep = pl.pallas_call(
    _edge_prep_body,
    grid=(_EB,),
    in_specs=[
        pl.BlockSpec((E // (8 * _EB), DF), lambda i: (i, 0)),
        pl.BlockSpec((DE, H), lambda i: (0, 0)),
        pl.BlockSpec((H, 1), lambda i: (0, 0)),
        pl.BlockSpec((DE, H), lambda i: (0, 0)),
        pl.BlockSpec((H, 1), lambda i: (0, 0)),
    ],
    out_specs=(pl.BlockSpec((E // (8 * _EB), 8), lambda i: (i, 0)),
               pl.BlockSpec((E // (8 * _EB), 8), lambda i: (i, 0))),
    out_shape=(jax.ShapeDtypeStruct((E // 8, 8), jnp.float32),
               jax.ShapeDtypeStruct((E // 8, 8), jnp.float32)),
)

_node_prep = pl.pallas_call(
    _node_prep_body,
    out_shape=(jax.ShapeDtypeStruct((NPAD, H), jnp.float32),
               jax.ShapeDtypeStruct((NPAD, 1), jnp.float32),
               jax.ShapeDtypeStruct((NPAD, 1), jnp.float32)),
)

_combine = pl.pallas_call(
    _combine_body,
    out_shape=(jax.ShapeDtypeStruct((NPAD, H), jnp.float32),
               jax.ShapeDtypeStruct((NPAD, 1), jnp.float32),
               jax.ShapeDtypeStruct((NPAD, 1), jnp.float32)),
)

_heads = pl.pallas_call(
    _heads_body,
    out_shape=(jax.ShapeDtypeStruct((1, NACT), jnp.float32),
               jax.ShapeDtypeStruct((N, 1), jnp.float32)),
)


def kernel(x, edge_attr, W1, We1, as1, ad1, ae1, b1, W2, We2, as2, ad2, ae2,
           b2, A1, bA1, A2, bA2, N1, bN1, N2, bN2, N3, bN3, edge_index):
    z2 = jnp.zeros((NPAD, H), jnp.float32)
    z1 = jnp.zeros((NPAD,), jnp.float32)

    ae_l1, ae_l2 = _edge_prep(edge_attr.reshape(E // 8, DF), We1,
                              ae1.reshape(H, 1), We2, ae2.reshape(H, 1))
    ae_l1 = ae_l1.reshape(E)
    ae_l2 = ae_l2.reshape(E)

    h1, asrc1, adst1 = _node_prep(x, W1, as1.reshape(H, 1), ad1.reshape(H, 1))
    out1, s1 = _gat_edge_sc(edge_index, ae_l1, asrc1.reshape(NPAD),
                            adst1.reshape(NPAD), h1, z2, z1)

    h2, asrc2, adst2 = _combine(out1, s1.reshape(2, NPAD, 1),
                                b1.reshape(1, H), W2,
                                as2.reshape(H, 1), ad2.reshape(H, 1))
    out2, s2 = _gat_edge_sc(edge_index, ae_l2, asrc2.reshape(NPAD),
                            adst2.reshape(NPAD), h2, z2, z1)

    action_prob, node_scores = _heads(
        out2, s2.reshape(2, NPAD, 1), b2.reshape(1, H),
        A1, bA1.reshape(1, H), A2, bA2.reshape(1, NACT),
        N1, bN1.reshape(1, H), N2, bN2.reshape(1, H),
        N3, bN3.reshape(1, 1))
    return (action_prob.reshape(NACT), node_scores)


# speedup vs baseline: 41.3781x; 1.0452x over previous
import functools

import jax
import jax.numpy as jnp
import numpy as np
from jax import lax
from jax.experimental import pallas as pl
from jax.experimental.pallas import tpu as pltpu
from jax.experimental.pallas import tpu_sc as plsc

N = 10000
E = 320000
DF = 128
DE = 16
H = 64
NACT = 3

NPAD = 10240
NW = 32
EW = E // NW
K = 80
NCH = EW // K
L = 16
RPT = NPAD // 16

_Z2 = np.zeros((NPAD, H), np.float32)


_sc_mesh = plsc.VectorSubcoreMesh(core_axis_name="c", subcore_axis_name="s")


@functools.partial(
    pl.kernel,
    out_type=(
        jax.ShapeDtypeStruct((2, NPAD, H), jnp.float32),
        jax.ShapeDtypeStruct((2, 16, NPAD), jnp.float32),
    ),
    mesh=_sc_mesh,
    compiler_params=pltpu.CompilerParams(needs_layout_passes=False,
                                         use_tc_tiling_on_sc=False),
    scratch_types=[
        pltpu.VMEM((EW,), jnp.int32),
        pltpu.VMEM((EW,), jnp.int32),
        pltpu.VMEM((EW,), jnp.float32),
        pltpu.VMEM((EW,), jnp.float32),
        pltpu.VMEM((NPAD,), jnp.float32),
        pltpu.VMEM((NPAD,), jnp.float32),
        pltpu.VMEM((NPAD,), jnp.float32),
        pltpu.VMEM((K,), jnp.int32),
        pltpu.VMEM((K,), jnp.int32),
        pltpu.VMEM((K,), jnp.int32),
        pltpu.VMEM((K, H), jnp.float32),
        pltpu.VMEM((K, H), jnp.float32),
        pltpu.VMEM_SHARED((NPAD, H), jnp.float32),
        pltpu.SemaphoreType.DMA,
        pltpu.SemaphoreType.DMA,
    ],
)
def _gat_edge_sc(ei_hbm, ae_hbm, asrc_hbm, adst_hbm, h_hbm,
                 z2_hbm, out_hbm, s_hbm,
                 src_v, dst_v, ae_v, e_v, asrc_v, adst_v, s_loc,
                 dstrow_v, srow0_v, srow1_v, rows0_v, rows1_v,
                 out_sh, sem0, sem1):
    cid = lax.axis_index("c")
    sid = lax.axis_index("s")
    wid = cid * 16 + sid
    b0 = wid * EW

    pltpu.sync_copy(ei_hbm.at[0, pl.ds(b0, EW)], src_v)
    pltpu.sync_copy(ei_hbm.at[1, pl.ds(b0, EW)], dst_v)
    pltpu.sync_copy(ae_hbm.at[pl.ds(b0, EW)], ae_v)
    pltpu.sync_copy(asrc_hbm, asrc_v)
    pltpu.sync_copy(adst_hbm, adst_v)

    r0 = sid * RPT
    pltpu.sync_copy(z2_hbm.at[pl.ds(r0, RPT)], out_sh.at[pl.ds(r0, RPT)])
    zero = jnp.zeros((L,), jnp.float32)

    def zloop(m, carry):
        s_loc[pl.ds(m * L, L)] = zero
        return carry

    lax.fori_loop(0, NPAD // L, zloop, 0)
    plsc.subcore_barrier()

    def p1(m, carry):
        sidx = src_v[pl.ds(m * L, L)]
        didx = dst_v[pl.ds(m * L, L)]
        ae = ae_v[pl.ds(m * L, L)]
        lg = (plsc.load_gather(asrc_v, [sidx])
              + plsc.load_gather(adst_v, [didx]) + ae)
        lg = jnp.where(lg >= 0.0, lg, 0.2 * lg)
        e = jnp.exp(lg)
        e_v[pl.ds(m * L, L)] = e
        plsc.addupdate_scatter(s_loc, [didx], e)
        return carry

    lax.fori_loop(0, EW // L, p1, 0)

    def _gather(j, srow_v, rows_v, sem):
        for v in range(K // L):
            srow_v[pl.ds(v * L, L)] = src_v[pl.ds(j * K + v * L, L)]
        pltpu.async_copy(h_hbm.at[srow_v], rows_v, sem)

    def _process(j, srow_v, rows_v, sem, prefetch):
        if prefetch is not None:
            pj, psrow_v, prows_v, psem = prefetch

            @pl.when(pj < NCH)
            def _():
                _gather(pj, psrow_v, prows_v, psem)

        pltpu.make_async_copy(h_hbm.at[srow_v], rows_v, sem).wait()
        for v in range(K // L):
            dstrow_v[pl.ds(v * L, L)] = dst_v[pl.ds(j * K + v * L, L)]
        for g in range(K // L):
            ev = e_v[pl.ds(j * K + g * L, L)]
            for t in range(L):
                i = g * L + t
                ei = ev[t]
                for q in range(H // L):
                    rows_v[i, pl.ds(q * L, L)] = (
                        rows_v[i, pl.ds(q * L, L)] * ei)
        pltpu.sync_copy(rows_v, out_sh.at[dstrow_v], add=True)

    _gather(0, srow0_v, rows0_v, sem0)

    def pair(p, carry):
        jj = p * 2
        _process(jj, srow0_v, rows0_v, sem0, (jj + 1, srow1_v, rows1_v, sem1))
        _process(jj + 1, srow1_v, rows1_v, sem1,
                 (jj + 2, srow0_v, rows0_v, sem0))
        return carry

    lax.fori_loop(0, NCH // 2, pair, 0)
    _process(NCH - 1, srow0_v, rows0_v, sem0, None)
    plsc.subcore_barrier()

    pltpu.sync_copy(out_sh.at[pl.ds(r0, RPT)], out_hbm.at[cid, pl.ds(r0, RPT)])
    pltpu.sync_copy(s_loc, s_hbm.at[cid, sid])


def _edge_prep_body(ea_ref, we1_ref, ae1_ref, we2_ref, ae2_ref, o_ref):
    ea = ea_ref[...]
    wv1 = jnp.dot(we1_ref[...], ae1_ref[...],
                  preferred_element_type=jnp.float32)
    wv2 = jnp.dot(we2_ref[...], ae2_ref[...],
                  preferred_element_type=jnp.float32)
    wvs = jnp.concatenate([wv1.T, wv2.T], axis=0)
    o_ref[...] = lax.dot_general(wvs, ea, (((1,), (1,)), ((), ())),
                                 preferred_element_type=jnp.float32)


def _node_prep_body(x_ref, w_ref, as_ref, ad_ref, h_ref, asrc_ref, adst_ref):
    h = jnp.dot(x_ref[...], w_ref[...], preferred_element_type=jnp.float32)
    h_ref[pl.ds(0, N), :] = h
    h_ref[pl.ds(N, NPAD - N), :] = jnp.zeros((NPAD - N, H), jnp.float32)
    asrc = jnp.dot(h, as_ref[...], preferred_element_type=jnp.float32)
    adst = jnp.dot(h, ad_ref[...], preferred_element_type=jnp.float32)
    asrc_ref[pl.ds(0, N), :] = asrc
    asrc_ref[pl.ds(N, NPAD - N), :] = jnp.zeros((NPAD - N, 1), jnp.float32)
    adst_ref[pl.ds(0, N), :] = adst
    adst_ref[pl.ds(N, NPAD - N), :] = jnp.zeros((NPAD - N, 1), jnp.float32)


def _combine_body(num_ref, s_ref, b_ref, w2_ref, as_ref, ad_ref,
                  h2_ref, asrc_ref, adst_ref):
    num = num_ref[0] + num_ref[1]
    s = jnp.sum(s_ref[...], axis=(0, 1)).reshape(NPAD, 1)
    h1 = jnp.maximum(num / (s + 1e-16) + b_ref[...], 0.0)
    h2 = jnp.dot(h1, w2_ref[...], preferred_element_type=jnp.float32)
    h2_ref[...] = h2
    asrc_ref[...] = jnp.dot(h2, as_ref[...], preferred_element_type=jnp.float32)
    adst_ref[...] = jnp.dot(h2, ad_ref[...], preferred_element_type=jnp.float32)


def _heads_body(num_ref, s_ref, b_ref, a1_ref, ba1_ref, a2_ref, ba2_ref,
                n1_ref, bn1_ref, n2_ref, bn2_ref, n3_ref, bn3_ref,
                ap_ref, ns_ref):
    num = num_ref[0, :N, :] + num_ref[1, :N, :]
    s = jnp.sum(s_ref[...], axis=(0, 1)).reshape(NPAD, 1)[:N, :]
    h = num / (s + 1e-16) + b_ref[...]
    emb = jnp.mean(h, axis=0, keepdims=True)
    a = jnp.dot(emb, a1_ref[...], preferred_element_type=jnp.float32) + ba1_ref[...]
    a = jnp.where(a >= 0.0, a, 0.01 * a)
    a = jnp.dot(a, a2_ref[...], preferred_element_type=jnp.float32) + ba2_ref[...]
    a = jnp.where(a >= 0.0, a, 0.01 * a)
    a = a - jnp.max(a, axis=-1, keepdims=True)
    ea = jnp.exp(a)
    ap_ref[...] = ea / jnp.sum(ea, axis=-1, keepdims=True)
    z = jnp.dot(h, n1_ref[...], preferred_element_type=jnp.float32) + bn1_ref[...]
    z = jnp.where(z >= 0.0, z, 0.01 * z)
    z = jnp.dot(z, n2_ref[...], preferred_element_type=jnp.float32) + bn2_ref[...]
    z = jnp.where(z >= 0.0, z, 0.01 * z)
    z = jnp.dot(z, n3_ref[...], preferred_element_type=jnp.float32) + bn3_ref[...]
    ns_ref[...] = 1.0 / (1.0 + jnp.exp(-z))


_EB = 25
_edge_prep = pl.pallas_call(
    _edge_prep_body,
    grid=(_EB,),
    in_specs=[
        pl.BlockSpec((E // _EB, DE), lambda i: (i, 0)),
        pl.BlockSpec((DE, H), lambda i: (0, 0)),
        pl.BlockSpec((H, 1), lambda i: (0, 0)),
        pl.BlockSpec((DE, H), lambda i: (0, 0)),
        pl.BlockSpec((H, 1), lambda i: (0, 0)),
    ],
    out_specs=pl.BlockSpec((2, E // _EB), lambda i: (0, i)),
    out_shape=jax.ShapeDtypeStruct((2, E), jnp.float32),
)

_node_prep = pl.pallas_call(
    _node_prep_body,
    out_shape=(jax.ShapeDtypeStruct((NPAD, H), jnp.float32),
               jax.ShapeDtypeStruct((NPAD, 1), jnp.float32),
               jax.ShapeDtypeStruct((NPAD, 1), jnp.float32)),
)

_combine = pl.pallas_call(
    _combine_body,
    out_shape=(jax.ShapeDtypeStruct((NPAD, H), jnp.float32),
               jax.ShapeDtypeStruct((NPAD, 1), jnp.float32),
               jax.ShapeDtypeStruct((NPAD, 1), jnp.float32)),
)

_heads = pl.pallas_call(
    _heads_body,
    out_shape=(jax.ShapeDtypeStruct((1, NACT), jnp.float32),
               jax.ShapeDtypeStruct((N, 1), jnp.float32)),
)


def kernel(x, edge_attr, W1, We1, as1, ad1, ae1, b1, W2, We2, as2, ad2, ae2,
           b2, A1, bA1, A2, bA2, N1, bN1, N2, bN2, N3, bN3, edge_index):
    z2 = jnp.asarray(_Z2)

    ae_lay = _edge_prep(edge_attr, We1, ae1.reshape(H, 1),
                        We2, ae2.reshape(H, 1))

    h1, asrc1, adst1 = _node_prep(x, W1, as1.reshape(H, 1), ad1.reshape(H, 1))
    out1, s1 = _gat_edge_sc(edge_index, ae_lay[0], asrc1.reshape(NPAD),
                            adst1.reshape(NPAD), h1, z2)

    h2, asrc2, adst2 = _combine(out1, s1, b1.reshape(1, H), W2,
                                as2.reshape(H, 1), ad2.reshape(H, 1))
    out2, s2 = _gat_edge_sc(edge_index, ae_lay[1], asrc2.reshape(NPAD),
                            adst2.reshape(NPAD), h2, z2)

    action_prob, node_scores = _heads(
        out2, s2, b2.reshape(1, H),
        A1, bA1.reshape(1, H), A2, bA2.reshape(1, NACT),
        N1, bN1.reshape(1, H), N2, bN2.reshape(1, H),
        N3, bN3.reshape(1, 1))
    return (action_prob.reshape(NACT), node_scores)


# speedup vs baseline: 43.4830x; 1.0509x over previous
import functools

import jax
import jax.numpy as jnp
import numpy as np
from jax import lax
from jax.experimental import pallas as pl
from jax.experimental.pallas import tpu as pltpu
from jax.experimental.pallas import tpu_sc as plsc

N = 10000
E = 320000
DF = 128
DE = 16
H = 64
NACT = 3

NPAD = 10240
NW = 32
EW = E // NW
K = 80
NCH = EW // K
L = 16
RPT = NPAD // 16

_Z2 = np.zeros((NPAD, H), np.float32)


_sc_mesh = plsc.VectorSubcoreMesh(core_axis_name="c", subcore_axis_name="s")


@functools.partial(
    pl.kernel,
    out_type=(
        jax.ShapeDtypeStruct((2, NPAD, H), jnp.float32),
        jax.ShapeDtypeStruct((2, 16, NPAD), jnp.float32),
    ),
    mesh=_sc_mesh,
    compiler_params=pltpu.CompilerParams(needs_layout_passes=False,
                                         use_tc_tiling_on_sc=False),
    scratch_types=[
        pltpu.VMEM((EW,), jnp.int32),
        pltpu.VMEM((EW,), jnp.int32),
        pltpu.VMEM((EW,), jnp.float32),
        pltpu.VMEM((EW,), jnp.float32),
        pltpu.VMEM((NPAD,), jnp.float32),
        pltpu.VMEM((NPAD,), jnp.float32),
        pltpu.VMEM((NPAD,), jnp.float32),
        pltpu.VMEM((K,), jnp.int32),
        pltpu.VMEM((K,), jnp.int32),
        pltpu.VMEM((K,), jnp.int32),
        pltpu.VMEM((K, H), jnp.float32),
        pltpu.VMEM((K, H), jnp.float32),
        pltpu.VMEM_SHARED((NPAD, H), jnp.float32),
        pltpu.SemaphoreType.DMA,
        pltpu.SemaphoreType.DMA,
    ],
)
def _gat_edge_sc(src_hbm, dst_hbm, ae_hbm, asrc_hbm, adst_hbm, h_hbm,
                 z2_hbm, out_hbm, s_hbm,
                 src_v, dst_v, ae_v, e_v, asrc_v, adst_v, s_loc,
                 dstrow_v, srow0_v, srow1_v, rows0_v, rows1_v,
                 out_sh, sem0, sem1):
    cid = lax.axis_index("c")
    sid = lax.axis_index("s")
    wid = cid * 16 + sid
    b0 = wid * EW

    pltpu.sync_copy(src_hbm.at[pl.ds(b0, EW)], src_v)
    pltpu.sync_copy(dst_hbm.at[pl.ds(b0, EW)], dst_v)
    pltpu.sync_copy(ae_hbm.at[pl.ds(b0, EW)], ae_v)
    pltpu.sync_copy(asrc_hbm, asrc_v)
    pltpu.sync_copy(adst_hbm, adst_v)

    r0 = sid * RPT
    pltpu.sync_copy(z2_hbm.at[pl.ds(r0, RPT)], out_sh.at[pl.ds(r0, RPT)])
    zero = jnp.zeros((L,), jnp.float32)

    def zloop(m, carry):
        s_loc[pl.ds(m * L, L)] = zero
        return carry

    lax.fori_loop(0, NPAD // L, zloop, 0)
    plsc.subcore_barrier()

    def p1(m, carry):
        sidx = src_v[pl.ds(m * L, L)]
        didx = dst_v[pl.ds(m * L, L)]
        ae = ae_v[pl.ds(m * L, L)]
        lg = (plsc.load_gather(asrc_v, [sidx])
              + plsc.load_gather(adst_v, [didx]) + ae)
        lg = jnp.where(lg >= 0.0, lg, 0.2 * lg)
        e = jnp.exp(lg)
        e_v[pl.ds(m * L, L)] = e
        plsc.addupdate_scatter(s_loc, [didx], e)
        return carry

    lax.fori_loop(0, EW // L, p1, 0)

    def _gather(j, srow_v, rows_v, sem):
        for v in range(K // L):
            srow_v[pl.ds(v * L, L)] = src_v[pl.ds(j * K + v * L, L)]
        pltpu.async_copy(h_hbm.at[srow_v], rows_v, sem)

    def _process(j, srow_v, rows_v, sem, prefetch):
        if prefetch is not None:
            pj, psrow_v, prows_v, psem = prefetch

            @pl.when(pj < NCH)
            def _():
                _gather(pj, psrow_v, prows_v, psem)

        pltpu.make_async_copy(h_hbm.at[srow_v], rows_v, sem).wait()
        for v in range(K // L):
            dstrow_v[pl.ds(v * L, L)] = dst_v[pl.ds(j * K + v * L, L)]
        for g in range(K // L):
            ev = e_v[pl.ds(j * K + g * L, L)]
            for t in range(L):
                i = g * L + t
                ei = ev[t]
                for q in range(H // L):
                    rows_v[i, pl.ds(q * L, L)] = (
                        rows_v[i, pl.ds(q * L, L)] * ei)
        pltpu.sync_copy(rows_v, out_sh.at[dstrow_v], add=True)

    _gather(0, srow0_v, rows0_v, sem0)

    def pair(p, carry):
        jj = p * 2
        _process(jj, srow0_v, rows0_v, sem0, (jj + 1, srow1_v, rows1_v, sem1))
        _process(jj + 1, srow1_v, rows1_v, sem1,
                 (jj + 2, srow0_v, rows0_v, sem0))
        return carry

    lax.fori_loop(0, NCH // 2, pair, 0)
    _process(NCH - 1, srow0_v, rows0_v, sem0, None)
    plsc.subcore_barrier()

    pltpu.sync_copy(out_sh.at[pl.ds(r0, RPT)], out_hbm.at[cid, pl.ds(r0, RPT)])
    pltpu.sync_copy(s_loc, s_hbm.at[cid, sid])


def _edge_prep_body(ea_ref, we1_ref, ae1_ref, we2_ref, ae2_ref, o_ref):
    ea = ea_ref[...]
    wv1 = jnp.dot(we1_ref[...], ae1_ref[...],
                  preferred_element_type=jnp.float32)
    wv2 = jnp.dot(we2_ref[...], ae2_ref[...],
                  preferred_element_type=jnp.float32)
    wvs = jnp.concatenate([wv1.T, wv2.T], axis=0)
    o_ref[...] = lax.dot_general(wvs, ea, (((1,), (1,)), ((), ())),
                                 preferred_element_type=jnp.float32)


def _repack_body(ei_ref, aeo_ref, src_ref, dst_ref, ae1_ref, ae2_ref):
    src_ref[...] = ei_ref[0, :]
    dst_ref[...] = ei_ref[1, :]
    ae1_ref[...] = aeo_ref[0, :]
    ae2_ref[...] = aeo_ref[1, :]


def _node_prep_body(x_ref, w_ref, as_ref, ad_ref, h_ref, asrc_ref, adst_ref):
    h = jnp.dot(x_ref[...], w_ref[...], preferred_element_type=jnp.float32)
    h_ref[pl.ds(0, N), :] = h
    h_ref[pl.ds(N, NPAD - N), :] = jnp.zeros((NPAD - N, H), jnp.float32)
    asrc = jnp.sum(h * as_ref[...], axis=1)
    adst = jnp.sum(h * ad_ref[...], axis=1)
    asrc_ref[pl.ds(0, N)] = asrc
    asrc_ref[pl.ds(N, NPAD - N)] = jnp.zeros((NPAD - N,), jnp.float32)
    adst_ref[pl.ds(0, N)] = adst
    adst_ref[pl.ds(N, NPAD - N)] = jnp.zeros((NPAD - N,), jnp.float32)


def _combine_body(num_ref, s_ref, b_ref, w2_ref, as_ref, ad_ref,
                  h2_ref, asrc_ref, adst_ref):
    num = num_ref[0] + num_ref[1]
    s = jnp.sum(s_ref[...], axis=(0, 1)).reshape(NPAD, 1)
    h1 = jnp.maximum(num / (s + 1e-16) + b_ref[...], 0.0)
    h2 = jnp.dot(h1, w2_ref[...], preferred_element_type=jnp.float32)
    h2_ref[...] = h2
    asrc_ref[...] = jnp.sum(h2 * as_ref[...], axis=1)
    adst_ref[...] = jnp.sum(h2 * ad_ref[...], axis=1)


def _heads_body(num_ref, s_ref, b_ref, a1_ref, ba1_ref, a2_ref, ba2_ref,
                n1_ref, bn1_ref, n2_ref, bn2_ref, n3_ref, bn3_ref,
                ap_ref, ns_ref):
    num = num_ref[0, :N, :] + num_ref[1, :N, :]
    s = jnp.sum(s_ref[...], axis=(0, 1)).reshape(NPAD, 1)[:N, :]
    h = num / (s + 1e-16) + b_ref[...]
    emb = jnp.mean(h, axis=0, keepdims=True)
    a = jnp.dot(emb, a1_ref[...], preferred_element_type=jnp.float32) + ba1_ref[...]
    a = jnp.where(a >= 0.0, a, 0.01 * a)
    a = jnp.dot(a, a2_ref[...], preferred_element_type=jnp.float32) + ba2_ref[...]
    a = jnp.where(a >= 0.0, a, 0.01 * a)
    a = a - jnp.max(a, axis=-1, keepdims=True)
    ea = jnp.exp(a)
    ap_ref[...] = ea / jnp.sum(ea, axis=-1, keepdims=True)
    z = jnp.dot(h, n1_ref[...], preferred_element_type=jnp.float32) + bn1_ref[...]
    z = jnp.where(z >= 0.0, z, 0.01 * z)
    z = jnp.dot(z, n2_ref[...], preferred_element_type=jnp.float32) + bn2_ref[...]
    z = jnp.where(z >= 0.0, z, 0.01 * z)
    z = jnp.dot(z, n3_ref[...], preferred_element_type=jnp.float32) + bn3_ref[...]
    ns_ref[...] = 1.0 / (1.0 + jnp.exp(-z))


_EB = 25
_edge_prep = pl.pallas_call(
    _edge_prep_body,
    grid=(_EB,),
    in_specs=[
        pl.BlockSpec((E // _EB, DE), lambda i: (i, 0)),
        pl.BlockSpec((DE, H), lambda i: (0, 0)),
        pl.BlockSpec((H, 1), lambda i: (0, 0)),
        pl.BlockSpec((DE, H), lambda i: (0, 0)),
        pl.BlockSpec((H, 1), lambda i: (0, 0)),
    ],
    out_specs=pl.BlockSpec((2, E // _EB), lambda i: (0, i)),
    out_shape=jax.ShapeDtypeStruct((2, E), jnp.float32),
)

_repack = pl.pallas_call(
    _repack_body,
    out_shape=(jax.ShapeDtypeStruct((E,), jnp.int32),
               jax.ShapeDtypeStruct((E,), jnp.int32),
               jax.ShapeDtypeStruct((E,), jnp.float32),
               jax.ShapeDtypeStruct((E,), jnp.float32)),
)

_node_prep = pl.pallas_call(
    _node_prep_body,
    out_shape=(jax.ShapeDtypeStruct((NPAD, H), jnp.float32),
               jax.ShapeDtypeStruct((NPAD,), jnp.float32),
               jax.ShapeDtypeStruct((NPAD,), jnp.float32)),
)

_combine = pl.pallas_call(
    _combine_body,
    out_shape=(jax.ShapeDtypeStruct((NPAD, H), jnp.float32),
               jax.ShapeDtypeStruct((NPAD,), jnp.float32),
               jax.ShapeDtypeStruct((NPAD,), jnp.float32)),
)

_heads = pl.pallas_call(
    _heads_body,
    out_shape=(jax.ShapeDtypeStruct((1, NACT), jnp.float32),
               jax.ShapeDtypeStruct((N, 1), jnp.float32)),
)


def kernel(x, edge_attr, W1, We1, as1, ad1, ae1, b1, W2, We2, as2, ad2, ae2,
           b2, A1, bA1, A2, bA2, N1, bN1, N2, bN2, N3, bN3, edge_index):
    z2 = jnp.asarray(_Z2)

    ae_lay = _edge_prep(edge_attr, We1, ae1.reshape(H, 1),
                        We2, ae2.reshape(H, 1))
    src1d, dst1d, ae_l1, ae_l2 = _repack(edge_index, ae_lay)

    h1, asrc1, adst1 = _node_prep(x, W1, as1.reshape(1, H), ad1.reshape(1, H))
    out1, s1 = _gat_edge_sc(src1d, dst1d, ae_l1, asrc1, adst1, h1, z2)

    h2, asrc2, adst2 = _combine(out1, s1, b1.reshape(1, H), W2,
                                as2.reshape(1, H), ad2.reshape(1, H))
    out2, s2 = _gat_edge_sc(src1d, dst1d, ae_l2, asrc2, adst2, h2, z2)

    action_prob, node_scores = _heads(
        out2, s2, b2.reshape(1, H),
        A1, bA1.reshape(1, H), A2, bA2.reshape(1, NACT),
        N1, bN1.reshape(1, H), N2, bN2.reshape(1, H),
        N3, bN3.reshape(1, 1))
    return (action_prob.reshape(NACT), node_scores)
